# baseline (speedup 1.0000x reference)
import functools

import jax
import jax.numpy as jnp
from jax import lax
from jax.experimental import pallas as pl
from jax.experimental.pallas import tpu as pltpu
from jax.experimental.pallas import tpu_sc as plsc

_N = 10000
_E = 320000
_D = 128
_NC = 2
_NS = 16
_NW = _NC * _NS
_CH = 128
_ROWS = _E // _CH
_CPT = _ROWS // _NW
_XTRA = _ROWS - _CPT * _NW
_NP = 10240
_RPT = _NP // _NS
_NSEC = 3
_SCH = _CPT // _NSEC

_mesh = plsc.VectorSubcoreMesh(
    core_axis_name="c", subcore_axis_name="s",
    num_cores=_NC, num_subcores=_NS)

_sc_params = pltpu.CompilerParams(use_tc_tiling_on_sc=False)
_sc_params_nl = pltpu.CompilerParams(use_tc_tiling_on_sc=False,
                                     needs_layout_passes=False)


def _zero_vmem(ref, rows, width):
    z = jnp.zeros((16,), jnp.float32)

    def body(i, _):
        for k in range(width // 16):
            ref[i, pl.ds(k * 16, 16)] = z
        return 0

    lax.fori_loop(0, rows, body, 0)


def _load_idx(src2d, dst2d, isrc_all, idst_all, w):
    pltpu.sync_copy(src2d.at[pl.ds(w * _CPT, _CPT)], isrc_all.at[pl.ds(0, _CPT)])
    pltpu.sync_copy(dst2d.at[pl.ds(w * _CPT, _CPT)], idst_all.at[pl.ds(0, _CPT)])

    @pl.when(w < _XTRA)
    def _():
        pltpu.sync_copy(src2d.at[pl.ds(_NW * _CPT + w, 1)],
                        isrc_all.at[pl.ds(_CPT, 1)])
        pltpu.sync_copy(dst2d.at[pl.ds(_NW * _CPT + w, 1)],
                        idst_all.at[pl.ds(_CPT, 1)])


@functools.partial(
    pl.kernel,
    out_type=jax.ShapeDtypeStruct((_NC, 2, _NP, 16), jnp.float32),
    mesh=_mesh,
    compiler_params=_sc_params,
    scratch_types=[
        pltpu.VMEM((_CPT + 1, _CH), jnp.int32),
        pltpu.VMEM((_CPT + 1, _CH), jnp.int32),
        pltpu.VMEM((_CH, 16), jnp.float32),
        pltpu.VMEM((_CH, 16), jnp.float32),
        pltpu.VMEM_SHARED((_NP, 16), jnp.float32),
        pltpu.VMEM_SHARED((_NP, 16), jnp.float32),
        pltpu.SemaphoreType.DMA,
        pltpu.SemaphoreType.DMA,
        pltpu.SemaphoreType.DMA,
        pltpu.SemaphoreType.DMA,
    ],
)
def _deg_kernel(src2d, dst2d, degp_hbm,
                isrc_all, idst_all, ones_v, zbuf, acc_s, acc_d,
                sem_sa, sem_da, sem_sb, sem_db):
    c = lax.axis_index("c")
    s = lax.axis_index("s")
    w = c * _NS + s

    _zero_vmem(zbuf, _CH, 16)
    one = jnp.ones((16,), jnp.float32)

    def fill_ones(i, _):
        ones_v[i] = one
        return 0

    lax.fori_loop(0, _CH, fill_ones, 0)

    for j in range(_RPT // _CH):
        pltpu.sync_copy(zbuf, acc_s.at[pl.ds(s * _RPT + j * _CH, _CH)])
        pltpu.sync_copy(zbuf, acc_d.at[pl.ds(s * _RPT + j * _CH, _CH)])
    plsc.subcore_barrier()

    _load_idx(src2d, dst2d, isrc_all, idst_all, w)

    def fire(t, ss, sd):
        pltpu.async_copy(ones_v, acc_s.at[isrc_all.at[t]], ss, add=True)
        pltpu.async_copy(ones_v, acc_d.at[idst_all.at[t]], sd, add=True)

    def drain(t, ss, sd):
        pltpu.make_async_copy(ones_v, acc_s.at[isrc_all.at[t]], ss).wait()
        pltpu.make_async_copy(ones_v, acc_d.at[idst_all.at[t]], sd).wait()

    fire(0, sem_sa, sem_da)

    def pair(g, _):
        t = 2 * g
        fire(t + 1, sem_sb, sem_db)
        drain(t, sem_sa, sem_da)

        @pl.when(t + 2 <= _CPT - 1)
        def _():
            fire(t + 2, sem_sa, sem_da)

        drain(t + 1, sem_sb, sem_db)
        return 0

    lax.fori_loop(0, _CPT // 2, pair, 0)

    @pl.when(w < _XTRA)
    def _():
        pltpu.sync_copy(ones_v, acc_s.at[isrc_all.at[_CPT]], add=True)
        pltpu.sync_copy(ones_v, acc_d.at[idst_all.at[_CPT]], add=True)

    plsc.subcore_barrier()
    pltpu.sync_copy(acc_s.at[pl.ds(s * _RPT, _RPT)],
                    degp_hbm.at[c, 0, pl.ds(s * _RPT, _RPT)])
    pltpu.sync_copy(acc_d.at[pl.ds(s * _RPT, _RPT)],
                    degp_hbm.at[c, 1, pl.ds(s * _RPT, _RPT)])


@functools.partial(
    pl.kernel,
    out_type=jax.ShapeDtypeStruct((_NC, _NP, _D), jnp.float32),
    mesh=_mesh,
    compiler_params=_sc_params,
    scratch_types=[
        pltpu.VMEM((_SCH + 1, _CH), jnp.int32),
        pltpu.VMEM((_SCH + 1, _CH), jnp.int32),
        pltpu.VMEM((_CH, _D), jnp.float32),
        pltpu.VMEM((_CH, _D), jnp.float32),
        pltpu.VMEM_SHARED((_NP, _D), jnp.float32),
        pltpu.SemaphoreType.DMA,
        pltpu.SemaphoreType.DMA,
    ],
)
def _agg_kernel(src2d, dst2d, g_hbm, out_hbm,
                isrc_sec, idst_sec, rows0, rows1, acc, sem_g0, sem_g1):
    c = lax.axis_index("c")
    s = lax.axis_index("s")
    w = c * _NS + s

    _zero_vmem(rows0, _CH, _D)
    for j in range(_RPT // _CH):
        pltpu.sync_copy(rows0, acc.at[pl.ds(s * _RPT + j * _CH, _CH)])
    plsc.subcore_barrier()

    for h in range(_NSEC):
        sb = w * _CPT + h * _SCH
        pltpu.sync_copy(src2d.at[pl.ds(sb, _SCH)], isrc_sec.at[pl.ds(0, _SCH)])
        pltpu.sync_copy(dst2d.at[pl.ds(sb, _SCH)], idst_sec.at[pl.ds(0, _SCH)])
        if h == _NSEC - 1:
            @pl.when(w < _XTRA)
            def _():
                pltpu.sync_copy(src2d.at[pl.ds(_NW * _CPT + w, 1)],
                                isrc_sec.at[pl.ds(_SCH, 1)])
                pltpu.sync_copy(dst2d.at[pl.ds(_NW * _CPT + w, 1)],
                                idst_sec.at[pl.ds(_SCH, 1)])

        pltpu.async_copy(g_hbm.at[isrc_sec.at[0]], rows0, sem_g0)

        def pair(g, _):
            t = 2 * g
            pltpu.async_copy(g_hbm.at[isrc_sec.at[t + 1]], rows1, sem_g1)
            pltpu.make_async_copy(g_hbm.at[isrc_sec.at[t]], rows0, sem_g0).wait()
            pltpu.sync_copy(rows0, acc.at[idst_sec.at[t]], add=True)

            @pl.when(t + 2 <= _SCH - 1)
            def _():
                pltpu.async_copy(g_hbm.at[isrc_sec.at[t + 2]], rows0, sem_g0)

            pltpu.make_async_copy(g_hbm.at[isrc_sec.at[t + 1]], rows1,
                                  sem_g1).wait()
            pltpu.sync_copy(rows1, acc.at[idst_sec.at[t + 1]], add=True)
            return 0

        lax.fori_loop(0, _SCH // 2, pair, 0)

    @pl.when(w < _XTRA)
    def _():
        pltpu.async_copy(g_hbm.at[isrc_sec.at[_SCH]], rows0, sem_g0).wait()
        pltpu.sync_copy(rows0, acc.at[idst_sec.at[_SCH]], add=True)

    plsc.subcore_barrier()
    pltpu.sync_copy(acc.at[pl.ds(s * _RPT, _RPT)],
                    out_hbm.at[c, pl.ds(s * _RPT, _RPT)])


_CW = _CH * 10


@functools.partial(
    pl.kernel,
    out_type=jax.ShapeDtypeStruct((_E * 10,), jnp.float32),
    mesh=_mesh,
    compiler_params=_sc_params_nl,
    scratch_types=[
        pltpu.VMEM((_CPT + 1, _CH), jnp.int32),
        pltpu.VMEM((_CPT + 1, _CH), jnp.int32),
        pltpu.VMEM((_CH, 16), jnp.float32),
        pltpu.VMEM((_CH, 16), jnp.float32),
        pltpu.VMEM((_CH, 16), jnp.float32),
        pltpu.VMEM((_CH, 16), jnp.float32),
        pltpu.VMEM((_CW,), jnp.float32),
        pltpu.VMEM((_CW,), jnp.float32),
        pltpu.VMEM((_CW // 16, 16), jnp.int32),
        pltpu.VMEM((_CW // 16, 16), jnp.int32),
        pltpu.VMEM_SHARED((_NP, 16), jnp.float32),
        pltpu.VMEM_SHARED((_NP, 16), jnp.float32),
        pltpu.SemaphoreType.DMA,
        pltpu.SemaphoreType.DMA,
        pltpu.SemaphoreType.DMA,
        pltpu.SemaphoreType.DMA,
        pltpu.SemaphoreType.DMA,
        pltpu.SemaphoreType.DMA,
    ],
)
def _score_kernel(src2d, dst2d, p_hbm, q_hbm, out_hbm,
                  isrc_all, idst_all, bufp0, bufq0, bufp1, bufq1,
                  cbuf0, cbuf1, rowtab, coltab,
                  p_sh, q_sh, sem_p0, sem_q0, sem_p1, sem_q1, sem_w0, sem_w1):
    c = lax.axis_index("c")
    s = lax.axis_index("s")
    w = c * _NS + s

    lanes = lax.iota(jnp.int32, 16)

    def mktab(j, _):
        p = j * 16 + lanes
        row = lax.shift_right_logical(p * 52429, 19)
        rowtab[j] = row
        coltab[j] = p - row * 10
        return 0

    lax.fori_loop(0, _CW // 16, mktab, 0)

    pltpu.sync_copy(p_hbm.at[pl.ds(s * _RPT, _RPT)],
                    p_sh.at[pl.ds(s * _RPT, _RPT)])
    pltpu.sync_copy(q_hbm.at[pl.ds(s * _RPT, _RPT)],
                    q_sh.at[pl.ds(s * _RPT, _RPT)])
    plsc.subcore_barrier()

    _load_idx(src2d, dst2d, isrc_all, idst_all, w)

    def fire(t, bp, bq, sp, sq):
        pltpu.async_copy(p_sh.at[isrc_all.at[t]], bp, sp)
        pltpu.async_copy(q_sh.at[idst_all.at[t]], bq, sq)

    def drain(t, bp, bq, sp, sq):
        pltpu.make_async_copy(p_sh.at[isrc_all.at[t]], bp, sp).wait()
        pltpu.make_async_copy(q_sh.at[idst_all.at[t]], bq, sq).wait()

    def add_chunk(bp, bq, cb):
        def body(i, _):
            for k in range(4):
                j = i * 4 + k
                rt, ct = rowtab[j], coltab[j]
                v = (plsc.load_gather(bp, [rt, ct])
                     + plsc.load_gather(bq, [rt, ct]))
                cb[pl.ds(j * 16, 16)] = v
            return 0

        lax.fori_loop(0, _CW // 64, body, 0)

    def out_slot(t):
        return out_hbm.at[pl.ds((w * _CPT + t) * _CW, _CW)]

    fire(0, bufp0, bufq0, sem_p0, sem_q0)
    fire(1, bufp1, bufq1, sem_p1, sem_q1)

    def pair(g, _):
        t = 2 * g
        drain(t, bufp0, bufq0, sem_p0, sem_q0)
        add_chunk(bufp0, bufq0, cbuf0)
        pltpu.async_copy(cbuf0, out_slot(t), sem_w0)
        drain(t + 1, bufp1, bufq1, sem_p1, sem_q1)
        add_chunk(bufp1, bufq1, cbuf1)
        pltpu.async_copy(cbuf1, out_slot(t + 1), sem_w1)

        @pl.when(t + 2 <= _CPT - 1)
        def _():
            pltpu.make_async_copy(cbuf0, out_slot(t), sem_w0).wait()
            fire(t + 2, bufp0, bufq0, sem_p0, sem_q0)

        @pl.when(t + 3 <= _CPT - 1)
        def _():
            pltpu.make_async_copy(cbuf1, out_slot(t + 1), sem_w1).wait()
            fire(t + 3, bufp1, bufq1, sem_p1, sem_q1)

        return 0

    lax.fori_loop(0, _CPT // 2, pair, 0)
    pltpu.make_async_copy(cbuf0, out_slot(_CPT - 2), sem_w0).wait()
    pltpu.make_async_copy(cbuf1, out_slot(_CPT - 1), sem_w1).wait()

    @pl.when(w < _XTRA)
    def _():
        pltpu.async_copy(p_sh.at[isrc_all.at[_CPT]], bufp0, sem_p0).wait()
        pltpu.async_copy(q_sh.at[idst_all.at[_CPT]], bufq0, sem_q0).wait()
        add_chunk(bufp0, bufq0, cbuf0)
        pltpu.sync_copy(cbuf0,
                        out_hbm.at[pl.ds((_NW * _CPT + w) * _CW, _CW)])


def _norms(degp):
    dsrc = degp[0, 0] + degp[1, 0]
    ddst = degp[0, 1] + degp[1, 1]
    ns = lax.rsqrt(jnp.clip(dsrc[:, 0:1], 1.0, None))
    nd = lax.rsqrt(jnp.clip(ddst[:, 0:1], 1.0, None))
    return ns, nd


def _dense1_body(x_ref, w_ref, degp_ref, o_ref):
    ns, _ = _norms(degp_ref[...])
    o_ref[...] = ns * jnp.dot(x_ref[...], w_ref[...],
                              preferred_element_type=jnp.float32)


_dense1 = pl.pallas_call(
    _dense1_body,
    out_shape=jax.ShapeDtypeStruct((_NP, _D), jnp.float32))


def _dense2_body(a_ref, degp_ref, w_ref, b_ref, o_ref):
    ns, nd = _norms(degp_ref[...])
    x1 = jax.nn.relu(nd * (a_ref[0] + a_ref[1]) + b_ref[...])
    o_ref[...] = ns * jnp.dot(x1, w_ref[...],
                              preferred_element_type=jnp.float32)


_dense2 = pl.pallas_call(
    _dense2_body,
    out_shape=jax.ShapeDtypeStruct((_NP, _D), jnp.float32))


def _dense3_body(a_ref, degp_ref, wp_ref, b_ref, bp_ref, p_ref, q_ref):
    _, nd = _norms(degp_ref[...])
    x2 = jax.nn.relu(nd * (a_ref[0] + a_ref[1]) + b_ref[...])
    p_ref[...] = jnp.dot(x2, wp_ref[0],
                         preferred_element_type=jnp.float32) + bp_ref[...]
    q_ref[...] = jnp.dot(x2, wp_ref[1],
                         preferred_element_type=jnp.float32)


_dense3 = pl.pallas_call(
    _dense3_body,
    out_shape=[jax.ShapeDtypeStruct((_NP, 16), jnp.float32),
               jax.ShapeDtypeStruct((_NP, 16), jnp.float32)])


def kernel(input_features, edge_index, sub_edge_index, W1, b1, W2, b2, Wp, bp):
    e_src = edge_index[0].astype(jnp.int32).reshape(_ROWS, _CH)
    e_dst = edge_index[1].astype(jnp.int32).reshape(_ROWS, _CH)
    s_src = sub_edge_index[0].astype(jnp.int32).reshape(_ROWS, _CH)
    s_dst = sub_edge_index[1].astype(jnp.int32).reshape(_ROWS, _CH)
    wp2 = jnp.zeros((2, _D, 16), jnp.float32)
    wp2 = wp2.at[0, :, :10].set(Wp[:_D])
    wp2 = wp2.at[1, :, :10].set(Wp[_D:])
    bp16 = jnp.zeros((1, 16), jnp.float32).at[0, :10].set(bp)

    xp = jnp.pad(input_features, ((0, _NP - _N), (0, 0)))
    degp = _deg_kernel(e_src, e_dst)
    g1 = _dense1(xp, W1, degp)
    a1 = _agg_kernel(e_src, e_dst, g1)
    g2 = _dense2(a1, degp, W2, b1.reshape(1, _D))
    a2 = _agg_kernel(e_src, e_dst, g2)
    p, q = _dense3(a2, degp, wp2, b2.reshape(1, _D), bp16)
    score_flat = _score_kernel(s_src, s_dst, p, q)
    return score_flat.reshape(_E, 10)


# speedup vs baseline: 1.2060x; 1.2060x over previous
import functools

import jax
import jax.numpy as jnp
from jax import lax
from jax.experimental import pallas as pl
from jax.experimental.pallas import tpu as pltpu
from jax.experimental.pallas import tpu_sc as plsc

_N = 10000
_E = 320000
_D = 128
_NC = 2
_NS = 16
_NW = _NC * _NS
_CH = 128
_ROWS = _E // _CH
_CPT = _ROWS // _NW
_XTRA = _ROWS - _CPT * _NW
_NP = 10240
_RPT = _NP // _NS
_NSEC = 3
_SCH = _CPT // _NSEC

_mesh = plsc.VectorSubcoreMesh(
    core_axis_name="c", subcore_axis_name="s",
    num_cores=_NC, num_subcores=_NS)

_sc_params = pltpu.CompilerParams(use_tc_tiling_on_sc=False)
_sc_params_nl = pltpu.CompilerParams(use_tc_tiling_on_sc=False,
                                     needs_layout_passes=False)


def _zero_vmem(ref, rows, width):
    z = jnp.zeros((16,), jnp.float32)

    def body(i, _):
        for k in range(width // 16):
            ref[i, pl.ds(k * 16, 16)] = z
        return 0

    lax.fori_loop(0, rows, body, 0)


def _load_idx(src2d, dst2d, isrc_all, idst_all, w):
    pltpu.sync_copy(src2d.at[pl.ds(w * _CPT, _CPT)], isrc_all.at[pl.ds(0, _CPT)])
    pltpu.sync_copy(dst2d.at[pl.ds(w * _CPT, _CPT)], idst_all.at[pl.ds(0, _CPT)])

    @pl.when(w < _XTRA)
    def _():
        pltpu.sync_copy(src2d.at[pl.ds(_NW * _CPT + w, 1)],
                        isrc_all.at[pl.ds(_CPT, 1)])
        pltpu.sync_copy(dst2d.at[pl.ds(_NW * _CPT + w, 1)],
                        idst_all.at[pl.ds(_CPT, 1)])


@functools.partial(
    pl.kernel,
    out_type=jax.ShapeDtypeStruct((_NC, 2, _NP, 16), jnp.float32),
    mesh=_mesh,
    compiler_params=_sc_params,
    scratch_types=[
        pltpu.VMEM((_CPT + 1, _CH), jnp.int32),
        pltpu.VMEM((_CPT + 1, _CH), jnp.int32),
        pltpu.VMEM((_CH, 16), jnp.float32),
        pltpu.VMEM((_CH, 16), jnp.float32),
        pltpu.VMEM_SHARED((_NP, 16), jnp.float32),
        pltpu.VMEM_SHARED((_NP, 16), jnp.float32),
        pltpu.SemaphoreType.DMA,
        pltpu.SemaphoreType.DMA,
        pltpu.SemaphoreType.DMA,
        pltpu.SemaphoreType.DMA,
    ],
)
def _deg_kernel(src2d, dst2d, degp_hbm,
                isrc_all, idst_all, ones_v, zbuf, acc_s, acc_d,
                sem_sa, sem_da, sem_sb, sem_db):
    c = lax.axis_index("c")
    s = lax.axis_index("s")
    w = c * _NS + s

    _zero_vmem(zbuf, _CH, 16)
    one = jnp.ones((16,), jnp.float32)

    def fill_ones(i, _):
        ones_v[i] = one
        return 0

    lax.fori_loop(0, _CH, fill_ones, 0)

    for j in range(_RPT // _CH):
        pltpu.sync_copy(zbuf, acc_s.at[pl.ds(s * _RPT + j * _CH, _CH)])
        pltpu.sync_copy(zbuf, acc_d.at[pl.ds(s * _RPT + j * _CH, _CH)])
    plsc.subcore_barrier()

    _load_idx(src2d, dst2d, isrc_all, idst_all, w)

    def fire(t, ss, sd):
        pltpu.async_copy(ones_v, acc_s.at[isrc_all.at[t]], ss, add=True)
        pltpu.async_copy(ones_v, acc_d.at[idst_all.at[t]], sd, add=True)

    def drain(t, ss, sd):
        pltpu.make_async_copy(ones_v, acc_s.at[isrc_all.at[t]], ss).wait()
        pltpu.make_async_copy(ones_v, acc_d.at[idst_all.at[t]], sd).wait()

    fire(0, sem_sa, sem_da)

    def pair(g, _):
        t = 2 * g
        fire(t + 1, sem_sb, sem_db)
        drain(t, sem_sa, sem_da)

        @pl.when(t + 2 <= _CPT - 1)
        def _():
            fire(t + 2, sem_sa, sem_da)

        drain(t + 1, sem_sb, sem_db)
        return 0

    lax.fori_loop(0, _CPT // 2, pair, 0)

    @pl.when(w < _XTRA)
    def _():
        pltpu.sync_copy(ones_v, acc_s.at[isrc_all.at[_CPT]], add=True)
        pltpu.sync_copy(ones_v, acc_d.at[idst_all.at[_CPT]], add=True)

    plsc.subcore_barrier()
    pltpu.sync_copy(acc_s.at[pl.ds(s * _RPT, _RPT)],
                    degp_hbm.at[c, 0, pl.ds(s * _RPT, _RPT)])
    pltpu.sync_copy(acc_d.at[pl.ds(s * _RPT, _RPT)],
                    degp_hbm.at[c, 1, pl.ds(s * _RPT, _RPT)])


@functools.partial(
    pl.kernel,
    out_type=jax.ShapeDtypeStruct((_NC, _NP, _D), jnp.float32),
    mesh=_mesh,
    compiler_params=_sc_params,
    scratch_types=[
        pltpu.VMEM((_SCH + 1, _CH), jnp.int32),
        pltpu.VMEM((_SCH + 1, _CH), jnp.int32),
        pltpu.VMEM((_CH, _D), jnp.float32),
        pltpu.VMEM((_CH, _D), jnp.float32),
        pltpu.VMEM_SHARED((_NP, _D), jnp.float32),
        pltpu.SemaphoreType.DMA,
        pltpu.SemaphoreType.DMA,
    ],
)
def _agg_kernel(src2d, dst2d, g_hbm, out_hbm,
                isrc_sec, idst_sec, rows0, rows1, acc, sem_g0, sem_g1):
    c = lax.axis_index("c")
    s = lax.axis_index("s")
    w = c * _NS + s

    _zero_vmem(rows0, _CH, _D)
    for j in range(_RPT // _CH):
        pltpu.sync_copy(rows0, acc.at[pl.ds(s * _RPT + j * _CH, _CH)])
    plsc.subcore_barrier()

    for h in range(_NSEC):
        sb = w * _CPT + h * _SCH
        pltpu.sync_copy(src2d.at[pl.ds(sb, _SCH)], isrc_sec.at[pl.ds(0, _SCH)])
        pltpu.sync_copy(dst2d.at[pl.ds(sb, _SCH)], idst_sec.at[pl.ds(0, _SCH)])
        if h == _NSEC - 1:
            @pl.when(w < _XTRA)
            def _():
                pltpu.sync_copy(src2d.at[pl.ds(_NW * _CPT + w, 1)],
                                isrc_sec.at[pl.ds(_SCH, 1)])
                pltpu.sync_copy(dst2d.at[pl.ds(_NW * _CPT + w, 1)],
                                idst_sec.at[pl.ds(_SCH, 1)])

        pltpu.async_copy(g_hbm.at[isrc_sec.at[0]], rows0, sem_g0)

        def pair(g, _):
            t = 2 * g
            pltpu.async_copy(g_hbm.at[isrc_sec.at[t + 1]], rows1, sem_g1)
            pltpu.make_async_copy(g_hbm.at[isrc_sec.at[t]], rows0, sem_g0).wait()
            pltpu.sync_copy(rows0, acc.at[idst_sec.at[t]], add=True)

            @pl.when(t + 2 <= _SCH - 1)
            def _():
                pltpu.async_copy(g_hbm.at[isrc_sec.at[t + 2]], rows0, sem_g0)

            pltpu.make_async_copy(g_hbm.at[isrc_sec.at[t + 1]], rows1,
                                  sem_g1).wait()
            pltpu.sync_copy(rows1, acc.at[idst_sec.at[t + 1]], add=True)
            return 0

        lax.fori_loop(0, _SCH // 2, pair, 0)

    @pl.when(w < _XTRA)
    def _():
        pltpu.async_copy(g_hbm.at[isrc_sec.at[_SCH]], rows0, sem_g0).wait()
        pltpu.sync_copy(rows0, acc.at[idst_sec.at[_SCH]], add=True)

    plsc.subcore_barrier()
    pltpu.sync_copy(acc.at[pl.ds(s * _RPT, _RPT)],
                    out_hbm.at[c, pl.ds(s * _RPT, _RPT)])


@functools.partial(
    pl.kernel,
    out_type=jax.ShapeDtypeStruct((_E, 16), jnp.float32),
    mesh=_mesh,
    compiler_params=_sc_params,
    scratch_types=[
        pltpu.VMEM((_CPT + 1, _CH), jnp.int32),
        pltpu.VMEM((_CPT + 1, _CH), jnp.int32),
        pltpu.VMEM((_CH, 16), jnp.float32),
        pltpu.VMEM((_CH, 16), jnp.float32),
        pltpu.VMEM((_CH, 16), jnp.float32),
        pltpu.VMEM((_CH, 16), jnp.float32),
        pltpu.VMEM_SHARED((_NP, 16), jnp.float32),
        pltpu.VMEM_SHARED((_NP, 16), jnp.float32),
        pltpu.SemaphoreType.DMA,
        pltpu.SemaphoreType.DMA,
        pltpu.SemaphoreType.DMA,
        pltpu.SemaphoreType.DMA,
        pltpu.SemaphoreType.DMA,
        pltpu.SemaphoreType.DMA,
    ],
)
def _score_kernel(src2d, dst2d, p_hbm, q_hbm, out_hbm,
                  isrc_all, idst_all, bufp0, bufq0, bufp1, bufq1,
                  p_sh, q_sh, sem_p0, sem_q0, sem_p1, sem_q1, sem_w0, sem_w1):
    c = lax.axis_index("c")
    s = lax.axis_index("s")
    w = c * _NS + s

    pltpu.sync_copy(p_hbm.at[pl.ds(s * _RPT, _RPT)],
                    p_sh.at[pl.ds(s * _RPT, _RPT)])
    pltpu.sync_copy(q_hbm.at[pl.ds(s * _RPT, _RPT)],
                    q_sh.at[pl.ds(s * _RPT, _RPT)])
    plsc.subcore_barrier()

    _load_idx(src2d, dst2d, isrc_all, idst_all, w)

    def fire(t, bp, bq, sp, sq):
        pltpu.async_copy(p_sh.at[isrc_all.at[t]], bp, sp)
        pltpu.async_copy(q_sh.at[idst_all.at[t]], bq, sq)

    def drain(t, bp, bq, sp, sq):
        pltpu.make_async_copy(p_sh.at[isrc_all.at[t]], bp, sp).wait()
        pltpu.make_async_copy(q_sh.at[idst_all.at[t]], bq, sq).wait()

    def add_chunk(bp, bq):
        def body(i, _):
            for k in range(4):
                r = i * 4 + k
                bp[r] = bp[r] + bq[r]
            return 0

        lax.fori_loop(0, _CH // 4, body, 0)

    def out_slot(t):
        return out_hbm.at[pl.ds((w * _CPT + t) * _CH, _CH)]

    fire(0, bufp0, bufq0, sem_p0, sem_q0)
    fire(1, bufp1, bufq1, sem_p1, sem_q1)

    def pair(g, _):
        t = 2 * g
        drain(t, bufp0, bufq0, sem_p0, sem_q0)
        add_chunk(bufp0, bufq0)
        pltpu.async_copy(bufp0, out_slot(t), sem_w0)
        drain(t + 1, bufp1, bufq1, sem_p1, sem_q1)
        add_chunk(bufp1, bufq1)
        pltpu.async_copy(bufp1, out_slot(t + 1), sem_w1)

        @pl.when(t + 2 <= _CPT - 1)
        def _():
            pltpu.make_async_copy(bufp0, out_slot(t), sem_w0).wait()
            fire(t + 2, bufp0, bufq0, sem_p0, sem_q0)

        @pl.when(t + 3 <= _CPT - 1)
        def _():
            pltpu.make_async_copy(bufp1, out_slot(t + 1), sem_w1).wait()
            fire(t + 3, bufp1, bufq1, sem_p1, sem_q1)

        return 0

    lax.fori_loop(0, _CPT // 2, pair, 0)
    pltpu.make_async_copy(bufp0, out_slot(_CPT - 2), sem_w0).wait()
    pltpu.make_async_copy(bufp1, out_slot(_CPT - 1), sem_w1).wait()

    @pl.when(w < _XTRA)
    def _():
        pltpu.async_copy(p_sh.at[isrc_all.at[_CPT]], bufp0, sem_p0).wait()
        pltpu.async_copy(q_sh.at[idst_all.at[_CPT]], bufq0, sem_q0).wait()
        add_chunk(bufp0, bufq0)
        pltpu.sync_copy(bufp0,
                        out_hbm.at[pl.ds((_NW * _CPT + w) * _CH, _CH)])


def _norms(degp):
    dsrc = degp[0, 0] + degp[1, 0]
    ddst = degp[0, 1] + degp[1, 1]
    ns = lax.rsqrt(jnp.clip(dsrc[:, 0:1], 1.0, None))
    nd = lax.rsqrt(jnp.clip(ddst[:, 0:1], 1.0, None))
    return ns, nd


def _dense1_body(x_ref, w_ref, degp_ref, o_ref):
    ns, _ = _norms(degp_ref[...])
    o_ref[...] = ns * jnp.dot(x_ref[...], w_ref[...],
                              preferred_element_type=jnp.float32)


_dense1 = pl.pallas_call(
    _dense1_body,
    out_shape=jax.ShapeDtypeStruct((_NP, _D), jnp.float32))


def _dense2_body(a_ref, degp_ref, w_ref, b_ref, o_ref):
    ns, nd = _norms(degp_ref[...])
    x1 = jax.nn.relu(nd * (a_ref[0] + a_ref[1]) + b_ref[...])
    o_ref[...] = ns * jnp.dot(x1, w_ref[...],
                              preferred_element_type=jnp.float32)


_dense2 = pl.pallas_call(
    _dense2_body,
    out_shape=jax.ShapeDtypeStruct((_NP, _D), jnp.float32))


def _dense3_body(a_ref, degp_ref, wp_ref, b_ref, bp_ref, p_ref, q_ref):
    _, nd = _norms(degp_ref[...])
    x2 = jax.nn.relu(nd * (a_ref[0] + a_ref[1]) + b_ref[...])
    p_ref[...] = jnp.dot(x2, wp_ref[0],
                         preferred_element_type=jnp.float32) + bp_ref[...]
    q_ref[...] = jnp.dot(x2, wp_ref[1],
                         preferred_element_type=jnp.float32)


_dense3 = pl.pallas_call(
    _dense3_body,
    out_shape=[jax.ShapeDtypeStruct((_NP, 16), jnp.float32),
               jax.ShapeDtypeStruct((_NP, 16), jnp.float32)])


def kernel(input_features, edge_index, sub_edge_index, W1, b1, W2, b2, Wp, bp):
    e_src = edge_index[0].astype(jnp.int32).reshape(_ROWS, _CH)
    e_dst = edge_index[1].astype(jnp.int32).reshape(_ROWS, _CH)
    s_src = sub_edge_index[0].astype(jnp.int32).reshape(_ROWS, _CH)
    s_dst = sub_edge_index[1].astype(jnp.int32).reshape(_ROWS, _CH)
    wp2 = jnp.zeros((2, _D, 16), jnp.float32)
    wp2 = wp2.at[0, :, :10].set(Wp[:_D])
    wp2 = wp2.at[1, :, :10].set(Wp[_D:])
    bp16 = jnp.zeros((1, 16), jnp.float32).at[0, :10].set(bp)

    xp = jnp.pad(input_features, ((0, _NP - _N), (0, 0)))
    degp = _deg_kernel(e_src, e_dst)
    g1 = _dense1(xp, W1, degp)
    a1 = _agg_kernel(e_src, e_dst, g1)
    g2 = _dense2(a1, degp, W2, b1.reshape(1, _D))
    a2 = _agg_kernel(e_src, e_dst, g2)
    p, q = _dense3(a2, degp, wp2, b2.reshape(1, _D), bp16)
    score16 = _score_kernel(s_src, s_dst, p, q)
    return score16[:, :10]
